# baseline (speedup 1.0000x reference)
import functools
from math import sqrt

import jax
import jax.numpy as jnp
from jax import lax
from jax.experimental import pallas as pl
from jax.experimental.pallas import tpu as pltpu
from jax.experimental.pallas import tpu_sc as plsc

VOCAB = 100000
DIM = 128
SCALE = float(sqrt(DIM))

NC = 2
NS = 16
NW = NC * NS

NBATCH = 4096
SEQ = 50

NSLICE = 4
SB = NBATCH // NSLICE
BPW = SB // NW
GB = 2
CHUNK = GB * SEQ
NCHUNK = BPW // GB

_mesh = plsc.VectorSubcoreMesh(core_axis_name="c", subcore_axis_name="s")


@functools.partial(
    pl.kernel,
    mesh=_mesh,
    out_type=jax.ShapeDtypeStruct((SB, SEQ, DIM), jnp.float32),
    scratch_types=[
        pltpu.VMEM((NCHUNK, CHUNK), jnp.int32),
        pltpu.VMEM((2, CHUNK, DIM), jnp.float32),
        pltpu.VMEM((2, GB, SEQ, DIM), jnp.float32),
        pltpu.SemaphoreType.DMA,
        pltpu.SemaphoreType.DMA,
    ],
)
def _gather_slice(idx_hbm, table_hbm, out_hbm, idx_v, gbuf, obuf, gsem, osem):
    wid = lax.axis_index("s") * NC + lax.axis_index("c")
    base = wid * BPW
    pltpu.sync_copy(idx_hbm.at[wid], idx_v)

    pltpu.async_copy(table_hbm.at[idx_v.at[0]], gbuf.at[0], gsem)
    pltpu.async_copy(table_hbm.at[idx_v.at[1]], gbuf.at[1], gsem)

    def pair_body(p, _):
        c0 = 2 * p
        for b in range(2):
            c = c0 + b
            pltpu.make_async_copy(
                table_hbm.at[idx_v.at[c]], gbuf.at[b], gsem).wait()

            @pl.when(c >= 2)
            def _wait_ocopy():
                pltpu.make_async_copy(
                    obuf.at[b],
                    out_hbm.at[pl.ds(base + (c - 2) * GB, GB)],
                    osem).wait()

            def scale_row(i, _):
                for bb in range(GB):
                    for jj in range(DIM // 16):
                        s = pl.ds(jj * 16, 16)
                        obuf[b, bb, i, s] = gbuf[b, bb * SEQ + i, s] * SCALE
                return 0

            lax.fori_loop(0, SEQ, scale_row, 0)

            @pl.when(c + 2 < NCHUNK)
            def _next_gather():
                pltpu.async_copy(
                    table_hbm.at[idx_v.at[c + 2]], gbuf.at[b], gsem)

            pltpu.async_copy(
                obuf.at[b], out_hbm.at[pl.ds(base + c * GB, GB)], osem)
        return 0

    lax.fori_loop(0, NCHUNK // 2, pair_body, 0)

    for b in range(2):
        c = NCHUNK - 2 + b
        pltpu.make_async_copy(
            obuf.at[b], out_hbm.at[pl.ds(base + c * GB, GB)],
            osem).wait()


def kernel(x, table):
    idx = x.reshape(NSLICE, NW, NCHUNK, CHUNK).astype(jnp.int32)
    parts = [_gather_slice(idx[s], table) for s in range(NSLICE)]
    return jnp.concatenate(parts, axis=0)


# speedup vs baseline: 1.0386x; 1.0386x over previous
import functools
from math import sqrt

import jax
import jax.numpy as jnp
from jax import lax
from jax.experimental import pallas as pl
from jax.experimental.pallas import tpu as pltpu
from jax.experimental.pallas import tpu_sc as plsc

VOCAB = 100000
DIM = 128
SCALE = float(sqrt(DIM))

NC = 2
NS = 16
NW = NC * NS

NBATCH = 4096
SEQ = 50

NSLICE = 4
SB = NBATCH // NSLICE
BPW = SB // NW
GB = 2
CHUNK = GB * SEQ
NCHUNK = BPW // GB

_mesh = plsc.VectorSubcoreMesh(core_axis_name="c", subcore_axis_name="s")


@functools.partial(
    pl.kernel,
    mesh=_mesh,
    out_type=jax.ShapeDtypeStruct((SB, SEQ, DIM), jnp.float32),
    scratch_types=[
        pltpu.VMEM((NCHUNK, CHUNK), jnp.int32),
        pltpu.VMEM((2, CHUNK, DIM), jnp.float32),
        pltpu.VMEM((2, GB, SEQ, DIM), jnp.float32),
        pltpu.SemaphoreType.DMA,
        pltpu.SemaphoreType.DMA,
    ],
)
def _gather_slice(idx_hbm, table_hbm, out_hbm, idx_v, gbuf, obuf, gsem, osem):
    wid = lax.axis_index("s") * NC + lax.axis_index("c")
    base = wid * BPW
    pltpu.sync_copy(idx_hbm.at[wid], idx_v)

    pltpu.async_copy(table_hbm.at[idx_v.at[0]], gbuf.at[0], gsem)
    pltpu.async_copy(table_hbm.at[idx_v.at[1]], gbuf.at[1], gsem)

    def pair_body(p, _):
        c0 = 2 * p
        for b in range(2):
            c = c0 + b
            pltpu.make_async_copy(
                table_hbm.at[idx_v.at[c]], gbuf.at[b], gsem).wait()

            @pl.when(c >= 2)
            def _wait_ocopy():
                pltpu.make_async_copy(
                    obuf.at[b],
                    out_hbm.at[pl.ds(base + (c - 2) * GB, GB)],
                    osem).wait()

            def scale_row(i, _):
                for bb in range(GB):
                    for jj in range(DIM // 16):
                        s = pl.ds(jj * 16, 16)
                        obuf[b, bb, i, s] = gbuf[b, bb * SEQ + i, s] * SCALE
                return 0

            lax.fori_loop(0, SEQ, scale_row, 0)

            @pl.when(c + 2 < NCHUNK)
            def _next_gather():
                pltpu.async_copy(
                    table_hbm.at[idx_v.at[c + 2]], gbuf.at[b], gsem)

            pltpu.async_copy(
                obuf.at[b], out_hbm.at[pl.ds(base + c * GB, GB)], osem)
        return 0

    lax.fori_loop(0, NCHUNK // 2, pair_body, 0)

    for b in range(2):
        c = NCHUNK - 2 + b
        pltpu.make_async_copy(
            obuf.at[b], out_hbm.at[pl.ds(base + c * GB, GB)],
            osem).wait()


def kernel(x, table):
    idx = x.reshape(NSLICE, NW, NCHUNK, CHUNK).astype(jnp.int32)
    parts = [_gather_slice(idx[s], table) for s in range(NSLICE)]
    out = jnp.zeros((NBATCH, SEQ, DIM), jnp.float32)
    for s in range(NSLICE):
        out = lax.dynamic_update_slice(out, parts[s], (s * SB, 0, 0))
        (out,) = lax.optimization_barrier((out,))
    return out


# speedup vs baseline: 1.1582x; 1.1152x over previous
import functools
from math import sqrt

import jax
import jax.numpy as jnp
from jax import lax
from jax.experimental import pallas as pl
from jax.experimental.pallas import tpu as pltpu
from jax.experimental.pallas import tpu_sc as plsc

VOCAB = 100000
DIM = 128
SCALE = float(sqrt(DIM))

NC = 2
NS = 16
NW = NC * NS

NBATCH = 4096
SEQ = 50
BPW = NBATCH // NW
GB = 2
CHUNK = GB * SEQ
NCHUNK = BPW // GB
RING = 4

_mesh = plsc.VectorSubcoreMesh(core_axis_name="c", subcore_axis_name="s")


@functools.partial(
    pl.kernel,
    mesh=_mesh,
    out_type=jax.ShapeDtypeStruct((NBATCH, SEQ, DIM), jnp.float32),
    scratch_types=[
        pltpu.VMEM((NCHUNK, CHUNK), jnp.int32),
        pltpu.VMEM((RING, CHUNK, DIM), jnp.float32),
        pltpu.SemaphoreType.DMA,
        pltpu.SemaphoreType.DMA,
    ],
)
def _gather_rows(idx_hbm, table_hbm, out_hbm, idx_v, gbuf, gsem, osem):
    wid = lax.axis_index("s") * NC + lax.axis_index("c")
    base = wid * BPW
    pltpu.sync_copy(idx_hbm.at[wid], idx_v)

    pltpu.async_copy(table_hbm.at[idx_v.at[0]], gbuf.at[0], gsem)
    pltpu.async_copy(table_hbm.at[idx_v.at[1]], gbuf.at[1], gsem)

    def _write(c, r):
        for bb in range(GB):
            pltpu.async_copy(
                gbuf.at[r, pl.ds(bb * SEQ, SEQ)],
                out_hbm.at[base + c * GB + bb], osem)

    def _wait_write(c, r):
        for bb in range(GB):
            pltpu.make_async_copy(
                gbuf.at[r, pl.ds(bb * SEQ, SEQ)],
                out_hbm.at[base + c * GB + bb], osem).wait()

    def group_body(p, _):
        c0 = RING * p
        for r in range(RING):
            c = c0 + r
            pltpu.make_async_copy(
                table_hbm.at[idx_v.at[c]], gbuf.at[r], gsem).wait()
            _write(c, r)

            @pl.when(c + 2 < NCHUNK)
            def _next_gather():
                @pl.when(c >= 2)
                def _drain():
                    _wait_write(c - 2, (r - 2) % RING)
                pltpu.async_copy(
                    table_hbm.at[idx_v.at[c + 2]], gbuf.at[(r + 2) % RING],
                    gsem)
        return 0

    lax.fori_loop(0, NCHUNK // RING, group_body, 0)

    for c in range(NCHUNK - 4, NCHUNK):
        _wait_write(c, c % RING)


def kernel(x, table):
    idx = x.reshape(NW, NCHUNK, CHUNK).astype(jnp.int32)
    return _gather_rows(idx, table) * SCALE


# speedup vs baseline: 1.1856x; 1.0237x over previous
import functools
from math import sqrt

import jax
import jax.numpy as jnp
from jax import lax
from jax.experimental import pallas as pl
from jax.experimental.pallas import tpu as pltpu
from jax.experimental.pallas import tpu_sc as plsc

VOCAB = 100000
DIM = 128
SCALE = float(sqrt(DIM))

NC = 2
NS = 16
NW = NC * NS

NBATCH = 4096
SEQ = 50
SEQ_PAD = 56

BPW = NBATCH // NW
GB = 2
CHUNK = GB * SEQ
NCHUNK = BPW // GB

_mesh = plsc.VectorSubcoreMesh(core_axis_name="c", subcore_axis_name="s")


@functools.partial(
    pl.kernel,
    mesh=_mesh,
    out_type=jax.ShapeDtypeStruct((NBATCH, SEQ_PAD, DIM), jnp.float32),
    scratch_types=[
        pltpu.VMEM((NCHUNK, CHUNK), jnp.int32),
        pltpu.VMEM((2, CHUNK, DIM), jnp.float32),
        pltpu.VMEM((2, GB, SEQ_PAD, DIM), jnp.float32),
        pltpu.SemaphoreType.DMA,
        pltpu.SemaphoreType.DMA,
    ],
)
def _gather_scale(idx_hbm, table_hbm, out_hbm, idx_v, gbuf, obuf, gsem, osem):
    wid = lax.axis_index("s") * NC + lax.axis_index("c")
    base = wid * BPW
    pltpu.sync_copy(idx_hbm.at[wid], idx_v)

    pltpu.async_copy(table_hbm.at[idx_v.at[0]], gbuf.at[0], gsem)
    pltpu.async_copy(table_hbm.at[idx_v.at[1]], gbuf.at[1], gsem)

    def pair_body(p, _):
        c0 = 2 * p
        for b in range(2):
            c = c0 + b
            pltpu.make_async_copy(
                table_hbm.at[idx_v.at[c]], gbuf.at[b], gsem).wait()

            @pl.when(c >= 2)
            def _wait_ocopy():
                pltpu.make_async_copy(
                    obuf.at[b],
                    out_hbm.at[pl.ds(base + (c - 2) * GB, GB)],
                    osem).wait()

            def scale_row(i, _):
                for bb in range(GB):
                    for jj in range(DIM // 16):
                        s = pl.ds(jj * 16, 16)
                        obuf[b, bb, i, s] = gbuf[b, bb * SEQ + i, s] * SCALE
                return 0

            lax.fori_loop(0, SEQ, scale_row, 0)

            @pl.when(c + 2 < NCHUNK)
            def _next_gather():
                pltpu.async_copy(
                    table_hbm.at[idx_v.at[c + 2]], gbuf.at[b], gsem)

            pltpu.async_copy(
                obuf.at[b], out_hbm.at[pl.ds(base + c * GB, GB)], osem)
        return 0

    lax.fori_loop(0, NCHUNK // 2, pair_body, 0)

    for b in range(2):
        c = NCHUNK - 2 + b
        pltpu.make_async_copy(
            obuf.at[b], out_hbm.at[pl.ds(base + c * GB, GB)],
            osem).wait()


BBK = 64


def _crop_body(in_ref, out_ref):
    out_ref[...] = in_ref[:, pl.ds(0, SEQ), :]


_crop = pl.pallas_call(
    _crop_body,
    grid=(NBATCH // BBK,),
    in_specs=[pl.BlockSpec((BBK, SEQ_PAD, DIM), lambda i: (i, 0, 0))],
    out_specs=pl.BlockSpec((BBK, SEQ, DIM), lambda i: (i, 0, 0)),
    out_shape=jax.ShapeDtypeStruct((NBATCH, SEQ, DIM), jnp.float32),
)


def kernel(x, table):
    idx = x.reshape(NW, NCHUNK, CHUNK).astype(jnp.int32)
    padded = _gather_scale(idx, table)
    return _crop(padded)


# speedup vs baseline: 1.7939x; 1.5131x over previous
import functools
from math import sqrt

import jax
import jax.numpy as jnp
from jax import lax
from jax.experimental import pallas as pl
from jax.experimental.pallas import tpu as pltpu
from jax.experimental.pallas import tpu_sc as plsc

VOCAB = 100000
DIM = 128
SCALE = float(sqrt(DIM))

NC = 2
NS = 16
NW = NC * NS

NBATCH = 4096
SEQ = 50
BPW = NBATCH // NW
GB = 2
CHUNK = GB * SEQ
NCHUNK = BPW // GB

_mesh = plsc.VectorSubcoreMesh(core_axis_name="c", subcore_axis_name="s")


@functools.partial(
    pl.kernel,
    mesh=_mesh,
    out_type=jax.ShapeDtypeStruct((NBATCH, SEQ, DIM), jnp.float32),
    scratch_types=[
        pltpu.VMEM((NCHUNK, CHUNK), jnp.int32),
        pltpu.VMEM((2, CHUNK, DIM), jnp.float32),
        pltpu.VMEM((2, GB, SEQ, DIM), jnp.float32),
        pltpu.SemaphoreType.DMA,
        pltpu.SemaphoreType.DMA,
    ],
)
def _gather_scale(idx_hbm, table_hbm, out_hbm, idx_v, gbuf, obuf, gsem, osem):
    wid = lax.axis_index("s") * NC + lax.axis_index("c")
    base = wid * BPW
    pltpu.sync_copy(idx_hbm.at[wid], idx_v)

    pltpu.async_copy(table_hbm.at[idx_v.at[0]], gbuf.at[0], gsem)
    pltpu.async_copy(table_hbm.at[idx_v.at[1]], gbuf.at[1], gsem)

    def pair_body(p, _):
        c0 = 2 * p
        for b in range(2):
            c = c0 + b
            pltpu.make_async_copy(
                table_hbm.at[idx_v.at[c]], gbuf.at[b], gsem).wait()

            @pl.when(c >= 2)
            def _wait_ocopy():
                pltpu.make_async_copy(
                    obuf.at[b],
                    out_hbm.at[pl.ds(base + (c - 2) * GB, GB)],
                    osem).wait()

            def scale_row(i, _):
                for bb in range(GB):
                    for jj in range(DIM // 16):
                        s = pl.ds(jj * 16, 16)
                        obuf[b, bb, i, s] = gbuf[b, bb * SEQ + i, s] * SCALE
                return 0

            lax.fori_loop(0, SEQ, scale_row, 0)

            @pl.when(c + 2 < NCHUNK)
            def _next_gather():
                pltpu.async_copy(
                    table_hbm.at[idx_v.at[c + 2]], gbuf.at[b], gsem)

            pltpu.async_copy(
                obuf.at[b], out_hbm.at[pl.ds(base + c * GB, GB)], osem)
        return 0

    lax.fori_loop(0, NCHUNK // 2, pair_body, 0)

    for b in range(2):
        c = NCHUNK - 2 + b
        pltpu.make_async_copy(
            obuf.at[b], out_hbm.at[pl.ds(base + c * GB, GB)],
            osem).wait()


def kernel(x, table):
    idx = x.reshape(NW, NCHUNK, CHUNK).astype(jnp.int32)
    return _gather_scale(idx, table)
